# Initial kernel scaffold; baseline (speedup 1.0000x reference)
import jax
import jax.numpy as jnp
from jax.experimental import pallas as pl


def kernel(x, table, pos_table):
    raise NotImplementedError("write your pallas kernel here")


# speedup vs baseline: 1.2701x; 1.2701x over previous
import functools

import jax
import jax.numpy as jnp
from jax import lax
from jax.experimental import pallas as pl
from jax.experimental.pallas import tpu as pltpu
from jax.experimental.pallas import tpu_sc as plsc

_B, _S, _D = 4, 2048, 128
_N = _B * _S
_NC, _NS = 2, 16
_NW = _NC * _NS
_CHUNK = _N // _NW
_GCH = 128
_NG = _CHUNK // _GCH

_mesh = plsc.VectorSubcoreMesh(core_axis_name="c", subcore_axis_name="s")


@functools.partial(
    pl.kernel,
    mesh=_mesh,
    out_type=jax.ShapeDtypeStruct((_N, _D), jnp.float32),
    scratch_types=[
        pltpu.VMEM((_NG, _GCH), jnp.int32),
        pltpu.VMEM((_CHUNK, _D), jnp.float32),
        pltpu.VMEM((_CHUNK, _D), jnp.float32),
        pltpu.SemaphoreType.DMA,
        pltpu.SemaphoreType.DMA,
    ],
)
def _emb(x_hbm, table_hbm, pos_hbm, out_hbm, idx_v, rows_v, pos_v, gsem, psem):
    wid = lax.axis_index("s") * _NC + lax.axis_index("c")
    base = wid * _CHUNK
    s0 = lax.rem(base, _S)
    pltpu.sync_copy(x_hbm.at[wid], idx_v)
    pcp = pltpu.async_copy(pos_hbm.at[pl.ds(s0, _CHUNK)], pos_v, psem)
    gcps = [
        pltpu.async_copy(
            table_hbm.at[idx_v.at[j]], rows_v.at[pl.ds(j * _GCH, _GCH)], gsem)
        for j in range(_NG)
    ]
    pcp.wait()
    for cp in gcps:
        cp.wait()

    def add_row(r, carry):
        for j in range(_D // 16):
            sl = pl.ds(j * 16, 16)
            rows_v[r, sl] = rows_v[r, sl] + pos_v[r, sl]
        return carry

    lax.fori_loop(0, _CHUNK, add_row, 0)
    pltpu.sync_copy(rows_v, out_hbm.at[pl.ds(base, _CHUNK)])


def kernel(x, table, pos_table):
    xw = x.reshape(_NW, _NG, _GCH).astype(jnp.int32)
    out = _emb(xw, table, pos_table)
    return out.reshape(_B, _S, _D)


# speedup vs baseline: 1.3301x; 1.0473x over previous
import functools

import jax
import jax.numpy as jnp
from jax import lax
from jax.experimental import pallas as pl
from jax.experimental.pallas import tpu as pltpu
from jax.experimental.pallas import tpu_sc as plsc

_B, _S, _D = 4, 2048, 128
_N = _B * _S
_NC, _NS = 2, 16
_NW = _NC * _NS
_CHUNK = _N // _NW
_GCH = 128
_NG = _CHUNK // _GCH

_mesh = plsc.VectorSubcoreMesh(core_axis_name="c", subcore_axis_name="s")


@functools.partial(
    pl.kernel,
    mesh=_mesh,
    out_type=jax.ShapeDtypeStruct((_N, _D), jnp.float32),
    scratch_types=[
        pltpu.VMEM((_NG, _GCH), jnp.int32),
        pltpu.VMEM((_CHUNK, _D), jnp.float32),
        pltpu.SemaphoreType.DMA,
    ],
)
def _emb(x_hbm, table_hbm, pos_hbm, out_hbm, idx_v, rows_v, gsem):
    wid = lax.axis_index("s") * _NC + lax.axis_index("c")
    base = wid * _CHUNK
    s0 = lax.rem(base, _S)
    pltpu.sync_copy(x_hbm.at[wid], idx_v)
    pltpu.sync_copy(pos_hbm.at[pl.ds(s0, _CHUNK)], rows_v)
    gcps = [
        pltpu.async_copy(
            table_hbm.at[idx_v.at[j]], rows_v.at[pl.ds(j * _GCH, _GCH)], gsem,
            add=True)
        for j in range(_NG)
    ]
    for cp in gcps:
        cp.wait()
    pltpu.sync_copy(rows_v, out_hbm.at[pl.ds(base, _CHUNK)])


def kernel(x, table, pos_table):
    xw = x.reshape(_NW, _NG, _GCH).astype(jnp.int32)
    out = _emb(xw, table, pos_table)
    return out.reshape(_B, _S, _D)


# speedup vs baseline: 1.3829x; 1.0397x over previous
import functools

import jax
import jax.numpy as jnp
from jax import lax
from jax.experimental import pallas as pl
from jax.experimental.pallas import tpu as pltpu
from jax.experimental.pallas import tpu_sc as plsc

_B, _S, _D = 4, 2048, 128
_N = _B * _S
_NC, _NS = 2, 16
_NW = _NC * _NS
_CHUNK = _N // _NW
_WPB = _S // _CHUNK
_GCH = 128
_NG = _CHUNK // _GCH

_mesh = plsc.VectorSubcoreMesh(core_axis_name="c", subcore_axis_name="s")


@functools.partial(
    pl.kernel,
    mesh=_mesh,
    out_type=jax.ShapeDtypeStruct((_B, _S, _D), jnp.float32),
    scratch_types=[
        pltpu.VMEM((_NG, _GCH), jnp.int32),
        pltpu.VMEM((_CHUNK, _D), jnp.float32),
        pltpu.SemaphoreType.DMA,
        pltpu.SemaphoreType.DMA,
        pltpu.SemaphoreType.DMA,
        pltpu.SemaphoreType.DMA,
        pltpu.SemaphoreType.DMA,
        pltpu.SemaphoreType.DMA,
    ],
)
def _emb(x_hbm, table_hbm, pos_hbm, out_hbm,
         idx_v, rows_v, isem, ps0, ps1, gs0, gs1, ws):
    wid = lax.axis_index("s") * _NC + lax.axis_index("c")
    b = wid // _WPB
    s0 = (wid % _WPB) * _CHUNK
    psems = (ps0, ps1)
    gsems = (gs0, gs1)

    icps = [
        pltpu.async_copy(
            x_hbm.at[b, pl.ds(s0 + j * _GCH, _GCH)], idx_v.at[j], isem)
        for j in range(_NG)
    ]
    pcps = [
        pltpu.async_copy(
            pos_hbm.at[pl.ds(s0 + j * _GCH, _GCH)],
            rows_v.at[pl.ds(j * _GCH, _GCH)], psems[j])
        for j in range(_NG)
    ]
    for cp in icps:
        cp.wait()

    gcps = []
    for j in range(_NG):
        pcps[j].wait()
        gcps.append(pltpu.async_copy(
            table_hbm.at[idx_v.at[j]], rows_v.at[pl.ds(j * _GCH, _GCH)],
            gsems[j], add=True))

    wcps = []
    for j in range(_NG):
        gcps[j].wait()
        wcps.append(pltpu.async_copy(
            rows_v.at[pl.ds(j * _GCH, _GCH)],
            out_hbm.at[b, pl.ds(s0 + j * _GCH, _GCH)], ws))
    for cp in wcps:
        cp.wait()


def kernel(x, table, pos_table):
    return _emb(x, table, pos_table)
